# baseline (speedup 1.0000x reference)
import functools

import jax
import jax.numpy as jnp
from jax.experimental import pallas as pl
from jax.experimental.pallas import tpu as pltpu

TOP_K = 8
BLOCK_M = 1024


def _router_body(x_ref, w_ref, dw_ref, idx_ref, aux_ref, util_acc, imp_acc,
                 *, n_experts):
    i = pl.program_id(0)
    n = pl.num_programs(0)

    h = jnp.dot(x_ref[...], w_ref[...], preferred_element_type=jnp.float32)
    ht = h.T

    expert = jax.lax.broadcasted_iota(jnp.int32, ht.shape, 0)
    work = ht
    topv = []
    topi = []
    for _ in range(TOP_K):
        m = jnp.max(work, axis=0, keepdims=True)
        eq = work == m
        idx = jnp.min(jnp.where(eq, expert, n_experts),
                      axis=0, keepdims=True)
        topv.append(m)
        topi.append(idx)
        work = jnp.where(expert == idx, -jnp.inf, work)

    vals = jnp.concatenate(topv, axis=0)
    idxs = jnp.concatenate(topi, axis=0)

    e = jnp.exp(vals - vals[0:1, :])
    dw = e / jnp.sum(e, axis=0, keepdims=True)
    eye = jnp.eye(TOP_K, dtype=jnp.float32)
    dn = (((0,), (0,)), ((), ()))
    dw_ref[...] = jax.lax.dot_general(dw, eye, dn,
                                      preferred_element_type=jnp.float32)
    idxf = jax.lax.dot_general(idxs.astype(jnp.float32), eye, dn,
                               preferred_element_type=jnp.float32)
    idx_ref[...] = idxf.astype(jnp.int32)

    p = jnp.exp(ht - vals[0:1, :])
    p = p / jnp.sum(p, axis=0, keepdims=True)
    imp_part = jnp.sum(p, axis=1, keepdims=True)
    util_part = jnp.sum(jnp.where(work == -jnp.inf, 1.0, 0.0),
                        axis=1, keepdims=True)

    @pl.when(i == 0)
    def _():
        util_acc[...] = jnp.zeros_like(util_acc)
        imp_acc[...] = jnp.zeros_like(imp_acc)

    util_acc[...] += util_part
    imp_acc[...] += imp_part

    @pl.when(i == n - 1)
    def _():
        def cv(v):
            mean = jnp.sum(v) / n_experts
            var = jnp.sum((v - mean) ** 2) / (n_experts - 1)
            return jnp.sqrt(var) / (mean + 1e-6)
        val = (cv(util_acc[...]) + cv(imp_acc[...])) * 0.01
        aux_ref[...] = jnp.full((1, 1), val, jnp.float32)


def kernel(x, W_gate, W_noise):
    orig_shape = x.shape
    d_model = x.shape[-1]
    n_experts = W_gate.shape[-1]
    xf = x.reshape(-1, d_model)
    t = xf.shape[0]
    bm = min(BLOCK_M, t)
    grid = t // bm

    dw_t, idx_t, aux = pl.pallas_call(
        functools.partial(_router_body, n_experts=n_experts),
        grid=(grid,),
        in_specs=[
            pl.BlockSpec((bm, d_model), lambda i: (i, 0)),
            pl.BlockSpec((d_model, n_experts), lambda i: (0, 0)),
        ],
        out_specs=[
            pl.BlockSpec((bm, TOP_K), lambda i: (i, 0)),
            pl.BlockSpec((bm, TOP_K), lambda i: (i, 0)),
            pl.BlockSpec((1, 1), lambda i: (0, 0)),
        ],
        out_shape=[
            jax.ShapeDtypeStruct((t, TOP_K), jnp.float32),
            jax.ShapeDtypeStruct((t, TOP_K), jnp.int32),
            jax.ShapeDtypeStruct((1, 1), jnp.float32),
        ],
        scratch_shapes=[
            pltpu.VMEM((n_experts, 1), jnp.float32),
            pltpu.VMEM((n_experts, 1), jnp.float32),
        ],
    )(xf, W_gate)

    return (dw_t.reshape(orig_shape[:-1] + (TOP_K,)),
            idx_t.reshape(orig_shape[:-1] + (TOP_K,)),
            aux[0, 0])


# speedup vs baseline: 1.1946x; 1.1946x over previous
import functools

import jax
import jax.numpy as jnp
from jax.experimental import pallas as pl
from jax.experimental.pallas import tpu as pltpu

TOP_K = 8
BLOCK_M = 1024


def _router_body(x_ref, w_ref, dw_ref, idx_ref, aux_ref, util_acc, imp_acc,
                 *, n_experts):
    i = pl.program_id(0)
    n = pl.num_programs(0)

    h = jnp.dot(x_ref[...], w_ref[...], preferred_element_type=jnp.float32)
    ht = h.T

    expert = jax.lax.broadcasted_iota(jnp.int32, ht.shape, 0)
    work = ht
    topv = []
    topi = []
    for _ in range(TOP_K):
        m = jnp.max(work, axis=0, keepdims=True)
        eq = work == m
        idx = jnp.min(jnp.where(eq, expert, n_experts),
                      axis=0, keepdims=True)
        topv.append(m)
        topi.append(idx)
        work = jnp.where(expert == idx, -jnp.inf, work)

    vals = jnp.concatenate(topv, axis=0)
    idxs = jnp.concatenate(topi, axis=0)

    e = jnp.exp(vals - vals[0:1, :])
    dw_ref[...] = e / jnp.sum(e, axis=0, keepdims=True)
    idx_ref[...] = idxs

    p = jnp.exp(ht - vals[0:1, :])
    p = p / jnp.sum(p, axis=0, keepdims=True)
    imp_part = jnp.sum(p, axis=1, keepdims=True)
    util_part = jnp.sum(jnp.where(work == -jnp.inf, 1.0, 0.0),
                        axis=1, keepdims=True)

    @pl.when(i == 0)
    def _():
        util_acc[...] = jnp.zeros_like(util_acc)
        imp_acc[...] = jnp.zeros_like(imp_acc)

    util_acc[...] += util_part
    imp_acc[...] += imp_part

    @pl.when(i == n - 1)
    def _():
        def cv(v):
            mean = jnp.sum(v) / n_experts
            var = jnp.sum((v - mean) ** 2) / (n_experts - 1)
            return jnp.sqrt(var) / (mean + 1e-6)
        val = (cv(util_acc[...]) + cv(imp_acc[...])) * 0.01
        aux_ref[...] = jnp.full((1, 1), val, jnp.float32)


def kernel(x, W_gate, W_noise):
    orig_shape = x.shape
    d_model = x.shape[-1]
    n_experts = W_gate.shape[-1]
    xf = x.reshape(-1, d_model)
    t = xf.shape[0]
    bm = min(BLOCK_M, t)
    grid = t // bm

    dw_t, idx_t, aux = pl.pallas_call(
        functools.partial(_router_body, n_experts=n_experts),
        grid=(grid,),
        in_specs=[
            pl.BlockSpec((bm, d_model), lambda i: (i, 0)),
            pl.BlockSpec((d_model, n_experts), lambda i: (0, 0)),
        ],
        out_specs=[
            pl.BlockSpec((TOP_K, bm), lambda i: (0, i)),
            pl.BlockSpec((TOP_K, bm), lambda i: (0, i)),
            pl.BlockSpec((1, 1), lambda i: (0, 0)),
        ],
        out_shape=[
            jax.ShapeDtypeStruct((TOP_K, t), jnp.float32),
            jax.ShapeDtypeStruct((TOP_K, t), jnp.int32),
            jax.ShapeDtypeStruct((1, 1), jnp.float32),
        ],
        scratch_shapes=[
            pltpu.VMEM((n_experts, 1), jnp.float32),
            pltpu.VMEM((n_experts, 1), jnp.float32),
        ],
    )(xf, W_gate)

    return (dw_t.T.reshape(orig_shape[:-1] + (TOP_K,)),
            idx_t.T.reshape(orig_shape[:-1] + (TOP_K,)),
            aux[0, 0])
